# baseline (speedup 1.0000x reference)
import jax
import jax.numpy as jnp
from jax import lax
from jax.experimental import pallas as pl
from jax.experimental.pallas import tpu as pltpu
from jax.experimental.pallas import tpu_sc as plsc

_N = 500000
_P = 256
_BP = 4096
_CSC = 1280
_SPLIT = _CSC * _P
_TCOFF = _SPLIT // _BP
_TCN = _N - _SPLIT


def _acos_poly(x):
    p = jnp.float32(-0.0012624911)
    for c in (0.0066700901, -0.0170881256, 0.0308918810, -0.0501743046,
              0.0889789874, -0.2145988016, 1.5707963050):
        p = p * x + jnp.float32(c)
    return p


def _sin_poly(y):
    y2 = y * y
    p = jnp.float32(1.0 / 362880.0)
    for c in (-1.0 / 5040.0, 1.0 / 120.0, -1.0 / 6.0, 1.0):
        p = p * y2 + jnp.float32(c)
    return y * p


def _rsqrt16(x):
    i = plsc.bitcast(x, jnp.int32)
    i = jnp.int32(0x5F3759DF) - lax.shift_right_logical(i, 1)
    y = plsc.bitcast(i, jnp.float32)
    y = y * (1.5 - 0.5 * x * y * y)
    y = y * (1.5 - 0.5 * x * y * y)
    return y


def _sqrt16(x):
    return x * _rsqrt16(x)


def _sc_body(xyz_hbm, rot_hbm, t_hbm, dt_hbm, out_hbm,
             xyz_a, rot_a, t_a, dt_a, out_a,
             xyz_b, rot_b, t_b, dt_b, out_b,
             sem_a, sem_b, sem_o):
    cid = lax.axis_index("c")
    sid = lax.axis_index("s")
    wid = sid * 2 + cid
    lane = lax.iota(jnp.int32, 16)

    def issue(i, bufs, sem):
        xv, rv, tv, dv = bufs[:4]
        base = pl.multiple_of(i * _P, _P)
        pltpu.async_copy(xyz_hbm.at[:, :, pl.ds(base, _P)], xv, sem)
        pltpu.async_copy(rot_hbm.at[:, :, pl.ds(base, _P)], rv, sem)
        pltpu.async_copy(t_hbm.at[pl.ds(base, _P)], tv, sem)
        pltpu.async_copy(dt_hbm.at[pl.ds(base, _P)], dv, sem)

    def wait_in(bufs, sem):
        xv, rv, tv, dv = bufs[:4]
        pltpu.make_async_copy(xyz_hbm.at[:, :, pl.ds(0, _P)], xv, sem).wait()
        pltpu.make_async_copy(rot_hbm.at[:, :, pl.ds(0, _P)], rv, sem).wait()
        pltpu.make_async_copy(t_hbm.at[pl.ds(0, _P)], tv, sem).wait()
        pltpu.make_async_copy(dt_hbm.at[pl.ds(0, _P)], dv, sem).wait()

    def process(i, bufs):
        xyz_v, rot_v, t_v, dt_v, out_v = bufs
        base = pl.multiple_of(i * _P, _P)

        def group(g, carry):
            g16 = g * 16
            pl16 = g16 + lane
            t16 = t_v[pl.ds(g16, 16)]
            dt16 = dt_v[pl.ds(g16, 16)]
            t16b = t16 + 1
            om = 1.0 - dt16
            for c in range(3):
                cc = jnp.full((16,), c, jnp.int32)
                a = plsc.load_gather(xyz_v, [cc, t16, pl16])
                b = plsc.load_gather(xyz_v, [cc, t16b, pl16])
                out_v[c, pl.ds(g16, 16)] = om * a + dt16 * b
            q1 = [plsc.load_gather(
                rot_v, [t16, jnp.full((16,), c, jnp.int32), pl16])
                for c in range(4)]
            q2 = [plsc.load_gather(
                rot_v, [t16b, jnp.full((16,), c, jnp.int32), pl16])
                for c in range(4)]
            n1 = q1[0] * q1[0] + q1[1] * q1[1] + q1[2] * q1[2] + q1[3] * q1[3]
            n2 = q2[0] * q2[0] + q2[1] * q2[1] + q2[2] * q2[2] + q2[3] * q2[3]
            d = q1[0] * q2[0] + q1[1] * q2[1] + q1[2] * q2[2] + q1[3] * q2[3]
            s1 = _rsqrt16(n1)
            s2 = _rsqrt16(n2)
            dotn = d * s1 * s2
            sgn = jnp.where(dotn < 0.0, jnp.float32(-1.0), jnp.float32(1.0))
            dotc = jnp.clip(jnp.abs(dotn), 0.0, 1.0 - 1e-7)
            omega = _sqrt16(jnp.maximum(1.0 - dotc, 1e-8)) * _acos_poly(dotc)
            so = _sqrt16(jnp.maximum(1.0 - dotc * dotc, 1e-8))
            w1 = _sin_poly(om * omega) / so
            w2 = _sin_poly(dt16 * omega) / so
            lerp_m = dotc > 0.9995
            W1 = jnp.where(lerp_m, om, w1)
            W2 = jnp.where(lerp_m, dt16, w2)
            A = W1 * s1
            B = W2 * s2 * sgn
            nr = _rsqrt16(A * A * n1 + 2.0 * A * B * d + B * B * n2)
            A = A * nr
            B = B * nr
            for c in range(4):
                out_v[3 + c, pl.ds(g16, 16)] = A * q1[c] + B * q2[c]
            return carry

        lax.fori_loop(0, _P // 16, group, 0)
        pltpu.async_copy(out_v, out_hbm.at[:, pl.ds(base, _P)], sem_o).wait()

    bufs_a = (xyz_a, rot_a, t_a, dt_a, out_a)
    bufs_b = (xyz_b, rot_b, t_b, dt_b, out_b)

    issue(wid, bufs_a, sem_a)

    def dbl(m, carry):
        i0 = wid + 32 * (2 * m)
        issue(i0 + 32, bufs_b, sem_b)
        wait_in(bufs_a, sem_a)
        process(i0, bufs_a)
        issue(i0 + 64, bufs_a, sem_a)
        wait_in(bufs_b, sem_b)
        process(i0 + 32, bufs_b)
        return carry

    lax.fori_loop(0, 19, dbl, 0)
    wait_in(bufs_a, sem_a)
    process(wid + 32 * 38, bufs_a)
    issue(wid + 32 * 39, bufs_b, sem_b)
    wait_in(bufs_b, sem_b)
    process(wid + 32 * 39, bufs_b)


def _sc_interp(xyzT, rotT, t1d, dt1d):
    f32 = jnp.float32
    i32 = jnp.int32
    mesh = plsc.VectorSubcoreMesh(core_axis_name="c", subcore_axis_name="s")
    buf = [
        pltpu.VMEM((3, 32, _P), f32),
        pltpu.VMEM((32, 4, _P), f32),
        pltpu.VMEM((_P,), i32),
        pltpu.VMEM((_P,), f32),
        pltpu.VMEM((8, _P), f32),
    ]
    call = pl.kernel(
        _sc_body,
        mesh=mesh,
        compiler_params=pltpu.CompilerParams(needs_layout_passes=False),
        out_type=jax.ShapeDtypeStruct((8, _SPLIT), f32),
        scratch_types=buf + buf + [
            pltpu.SemaphoreType.DMA,
            pltpu.SemaphoreType.DMA,
            pltpu.SemaphoreType.DMA,
        ],
    )
    return call(xyzT, rotT, t1d, dt1d)


def _tc_body(dt_ref, t_ref, xyz_ref, rot_ref, out_ref):
    dt = dt_ref[...]
    t = t_ref[...]
    t3 = t[:, None, :]
    xyz = xyz_ref[...]
    kx = lax.broadcasted_iota(jnp.int32, (1, 32, 1), 1)
    m1 = kx == t3
    m2 = kx == (t3 + 1)
    y1 = jnp.sum(jnp.where(m1, xyz, 0.0), axis=1)
    y2 = jnp.sum(jnp.where(m2, xyz, 0.0), axis=1)
    y = (1.0 - dt) * y1 + dt * y2
    rot = rot_ref[...]
    kr = lax.broadcasted_iota(jnp.int32, (32, 1, 1), 0)
    r1 = kr == t3
    r2 = kr == (t3 + 1)
    q1 = jnp.sum(jnp.where(r1, rot, 0.0), axis=0)
    q2 = jnp.sum(jnp.where(r2, rot, 0.0), axis=0)
    n1 = jnp.sum(q1 * q1, axis=0, keepdims=True)
    n2 = jnp.sum(q2 * q2, axis=0, keepdims=True)
    d = jnp.sum(q1 * q2, axis=0, keepdims=True)
    s1 = lax.rsqrt(n1)
    s2 = lax.rsqrt(n2)
    dotn = d * s1 * s2
    sign = jnp.where(dotn < 0.0, -1.0, 1.0)
    dotc = jnp.clip(jnp.abs(dotn), 0.0, 1.0 - 1e-7)
    omega = jnp.sqrt(jnp.maximum(1.0 - dotc, 0.0)) * _acos_poly(dotc)
    so = jnp.sqrt(1.0 - dotc * dotc)
    safe_so = jnp.where(so < 1e-6, 1.0, so)
    w1 = jnp.sin((1.0 - dt) * omega) / safe_so
    w2 = jnp.sin(dt * omega) / safe_so
    use_lerp = dotc > 0.9995
    W1 = jnp.where(use_lerp, 1.0 - dt, w1)
    W2 = jnp.where(use_lerp, dt, w2)
    A = W1 * s1
    B = W2 * s2 * sign
    nrm = lax.rsqrt(A * A * n1 + 2.0 * A * B * d + B * B * n2)
    A = A * nrm
    B = B * nrm
    out_ref[...] = jnp.concatenate([y, A * q1 + B * q2], axis=0)


def _tc_sweep(dtT, tT, xyzT, rotT):
    grid = pl.cdiv(_TCN, _BP)
    return pl.pallas_call(
        _tc_body,
        grid=(grid,),
        in_specs=[
            pl.BlockSpec((1, _BP), lambda i: (0, i + _TCOFF)),
            pl.BlockSpec((1, _BP), lambda i: (0, i + _TCOFF)),
            pl.BlockSpec((3, 32, _BP), lambda i: (0, 0, i + _TCOFF)),
            pl.BlockSpec((32, 4, _BP), lambda i: (0, 0, i + _TCOFF)),
        ],
        out_specs=pl.BlockSpec((7, _BP), lambda i: (0, i)),
        out_shape=jax.ShapeDtypeStruct((7, _TCN), jnp.float32),
    )(dtT, tT, xyzT, rotT)


def kernel(xyz_motion, rotation_motion, t_idx, delta_t):
    N = xyz_motion.shape[0]
    xyzT = xyz_motion.transpose(2, 1, 0)
    rotT = rotation_motion.transpose(1, 2, 0)
    t1d = t_idx.reshape(N)
    dt1d = delta_t.reshape(N)
    sc_out = _sc_interp(xyzT, rotT, t1d, dt1d)
    tc_out = _tc_sweep(delta_t.reshape(1, N), t_idx.reshape(1, N), xyzT, rotT)
    out = jnp.concatenate([sc_out[:7], tc_out], axis=1)
    return out.transpose(1, 0)


# speedup vs baseline: 1.0624x; 1.0624x over previous
import jax
import jax.numpy as jnp
from jax import lax
from jax.experimental import pallas as pl
from jax.experimental.pallas import tpu as pltpu
from jax.experimental.pallas import tpu_sc as plsc

_N = 500000
_P = 256
_BP = 4096
_CSC = 1408
_SPLIT = _CSC * _P
_TCOFF = _SPLIT // _BP
_TCN = _N - _SPLIT


def _acos_poly(x):
    p = jnp.float32(-0.0012624911)
    for c in (0.0066700901, -0.0170881256, 0.0308918810, -0.0501743046,
              0.0889789874, -0.2145988016, 1.5707963050):
        p = p * x + jnp.float32(c)
    return p


def _sin_poly(y):
    y2 = y * y
    p = jnp.float32(1.0 / 362880.0)
    for c in (-1.0 / 5040.0, 1.0 / 120.0, -1.0 / 6.0, 1.0):
        p = p * y2 + jnp.float32(c)
    return y * p


def _rsqrt16(x):
    i = plsc.bitcast(x, jnp.int32)
    i = jnp.int32(0x5F3759DF) - lax.shift_right_logical(i, 1)
    y = plsc.bitcast(i, jnp.float32)
    y = y * (1.5 - 0.5 * x * y * y)
    y = y * (1.5 - 0.5 * x * y * y)
    return y


def _sqrt16(x):
    return x * _rsqrt16(x)


def _sc_body(xyz_hbm, rot_hbm, t_hbm, dt_hbm, out_hbm,
             xyz_a, rot_a, t_a, dt_a, out_a,
             xyz_b, rot_b, t_b, dt_b, out_b,
             sem_a, sem_b, sem_o):
    cid = lax.axis_index("c")
    sid = lax.axis_index("s")
    wid = sid * 2 + cid
    lane = lax.iota(jnp.int32, 16)

    def issue(i, bufs, sem):
        xv, rv, tv, dv = bufs[:4]
        base = pl.multiple_of(i * _P, _P)
        pltpu.async_copy(xyz_hbm.at[:, :, pl.ds(base, _P)], xv, sem)
        pltpu.async_copy(rot_hbm.at[:, :, pl.ds(base, _P)], rv, sem)
        pltpu.async_copy(t_hbm.at[pl.ds(base, _P)], tv, sem)
        pltpu.async_copy(dt_hbm.at[pl.ds(base, _P)], dv, sem)

    def wait_in(bufs, sem):
        xv, rv, tv, dv = bufs[:4]
        pltpu.make_async_copy(xyz_hbm.at[:, :, pl.ds(0, _P)], xv, sem).wait()
        pltpu.make_async_copy(rot_hbm.at[:, :, pl.ds(0, _P)], rv, sem).wait()
        pltpu.make_async_copy(t_hbm.at[pl.ds(0, _P)], tv, sem).wait()
        pltpu.make_async_copy(dt_hbm.at[pl.ds(0, _P)], dv, sem).wait()

    def process(i, bufs):
        xyz_v, rot_v, t_v, dt_v, out_v = bufs
        base = pl.multiple_of(i * _P, _P)

        def group(g, carry):
            g16 = g * 16
            pl16 = g16 + lane
            t16 = t_v[pl.ds(g16, 16)]
            dt16 = dt_v[pl.ds(g16, 16)]
            t16b = t16 + 1
            om = 1.0 - dt16
            for c in range(3):
                cc = jnp.full((16,), c, jnp.int32)
                a = plsc.load_gather(xyz_v, [cc, t16, pl16])
                b = plsc.load_gather(xyz_v, [cc, t16b, pl16])
                out_v[c, pl.ds(g16, 16)] = om * a + dt16 * b
            q1 = [plsc.load_gather(
                rot_v, [t16, jnp.full((16,), c, jnp.int32), pl16])
                for c in range(4)]
            q2 = [plsc.load_gather(
                rot_v, [t16b, jnp.full((16,), c, jnp.int32), pl16])
                for c in range(4)]
            n1 = q1[0] * q1[0] + q1[1] * q1[1] + q1[2] * q1[2] + q1[3] * q1[3]
            n2 = q2[0] * q2[0] + q2[1] * q2[1] + q2[2] * q2[2] + q2[3] * q2[3]
            d = q1[0] * q2[0] + q1[1] * q2[1] + q1[2] * q2[2] + q1[3] * q2[3]
            s1 = _rsqrt16(n1)
            s2 = _rsqrt16(n2)
            dotn = d * s1 * s2
            sgn = jnp.where(dotn < 0.0, jnp.float32(-1.0), jnp.float32(1.0))
            dotc = jnp.clip(jnp.abs(dotn), 0.0, 1.0 - 1e-7)
            omega = _sqrt16(jnp.maximum(1.0 - dotc, 1e-8)) * _acos_poly(dotc)
            so = _sqrt16(jnp.maximum(1.0 - dotc * dotc, 1e-8))
            w1 = _sin_poly(om * omega) / so
            w2 = _sin_poly(dt16 * omega) / so
            lerp_m = dotc > 0.9995
            W1 = jnp.where(lerp_m, om, w1)
            W2 = jnp.where(lerp_m, dt16, w2)
            A = W1 * s1
            B = W2 * s2 * sgn
            nr = _rsqrt16(A * A * n1 + 2.0 * A * B * d + B * B * n2)
            A = A * nr
            B = B * nr
            for c in range(4):
                out_v[3 + c, pl.ds(g16, 16)] = A * q1[c] + B * q2[c]
            return carry

        lax.fori_loop(0, _P // 16, group, 0)
        pltpu.async_copy(out_v, out_hbm.at[:, pl.ds(base, _P)], sem_o).wait()

    bufs_a = (xyz_a, rot_a, t_a, dt_a, out_a)
    bufs_b = (xyz_b, rot_b, t_b, dt_b, out_b)

    issue(wid, bufs_a, sem_a)

    def dbl(m, carry):
        i0 = wid + 32 * (2 * m)
        issue(i0 + 32, bufs_b, sem_b)
        wait_in(bufs_a, sem_a)
        process(i0, bufs_a)
        issue(i0 + 64, bufs_a, sem_a)
        wait_in(bufs_b, sem_b)
        process(i0 + 32, bufs_b)
        return carry

    lax.fori_loop(0, 21, dbl, 0)
    wait_in(bufs_a, sem_a)
    process(wid + 32 * 42, bufs_a)
    issue(wid + 32 * 43, bufs_b, sem_b)
    wait_in(bufs_b, sem_b)
    process(wid + 32 * 43, bufs_b)


def _sc_interp(xyzT, rotT, t1d, dt1d):
    f32 = jnp.float32
    i32 = jnp.int32
    mesh = plsc.VectorSubcoreMesh(core_axis_name="c", subcore_axis_name="s")
    buf = [
        pltpu.VMEM((3, 32, _P), f32),
        pltpu.VMEM((32, 4, _P), f32),
        pltpu.VMEM((_P,), i32),
        pltpu.VMEM((_P,), f32),
        pltpu.VMEM((8, _P), f32),
    ]
    call = pl.kernel(
        _sc_body,
        mesh=mesh,
        compiler_params=pltpu.CompilerParams(needs_layout_passes=False),
        out_type=jax.ShapeDtypeStruct((8, _SPLIT), f32),
        scratch_types=buf + buf + [
            pltpu.SemaphoreType.DMA,
            pltpu.SemaphoreType.DMA,
            pltpu.SemaphoreType.DMA,
        ],
    )
    return call(xyzT, rotT, t1d, dt1d)


def _tc_body(dt_ref, t_ref, xyz_ref, rot_ref, out_ref):
    dt = dt_ref[...]
    t = t_ref[...]
    t3 = t[:, None, :]
    xyz = xyz_ref[...]
    kx = lax.broadcasted_iota(jnp.int32, (1, 32, 1), 1)
    m1 = kx == t3
    m2 = kx == (t3 + 1)
    y1 = jnp.sum(jnp.where(m1, xyz, 0.0), axis=1)
    y2 = jnp.sum(jnp.where(m2, xyz, 0.0), axis=1)
    y = (1.0 - dt) * y1 + dt * y2
    rot = rot_ref[...]
    kr = lax.broadcasted_iota(jnp.int32, (32, 1, 1), 0)
    r1 = kr == t3
    r2 = kr == (t3 + 1)
    q1 = jnp.sum(jnp.where(r1, rot, 0.0), axis=0)
    q2 = jnp.sum(jnp.where(r2, rot, 0.0), axis=0)
    n1 = jnp.sum(q1 * q1, axis=0, keepdims=True)
    n2 = jnp.sum(q2 * q2, axis=0, keepdims=True)
    d = jnp.sum(q1 * q2, axis=0, keepdims=True)
    s1 = lax.rsqrt(n1)
    s2 = lax.rsqrt(n2)
    dotn = d * s1 * s2
    sign = jnp.where(dotn < 0.0, -1.0, 1.0)
    dotc = jnp.clip(jnp.abs(dotn), 0.0, 1.0 - 1e-7)
    omega = jnp.sqrt(jnp.maximum(1.0 - dotc, 0.0)) * _acos_poly(dotc)
    so = jnp.sqrt(1.0 - dotc * dotc)
    safe_so = jnp.where(so < 1e-6, 1.0, so)
    w1 = jnp.sin((1.0 - dt) * omega) / safe_so
    w2 = jnp.sin(dt * omega) / safe_so
    use_lerp = dotc > 0.9995
    W1 = jnp.where(use_lerp, 1.0 - dt, w1)
    W2 = jnp.where(use_lerp, dt, w2)
    A = W1 * s1
    B = W2 * s2 * sign
    nrm = lax.rsqrt(A * A * n1 + 2.0 * A * B * d + B * B * n2)
    A = A * nrm
    B = B * nrm
    out_ref[...] = jnp.concatenate([y, A * q1 + B * q2], axis=0)


def _tc_sweep(dtT, tT, xyzT, rotT):
    grid = pl.cdiv(_TCN, _BP)
    return pl.pallas_call(
        _tc_body,
        grid=(grid,),
        in_specs=[
            pl.BlockSpec((1, _BP), lambda i: (0, i + _TCOFF)),
            pl.BlockSpec((1, _BP), lambda i: (0, i + _TCOFF)),
            pl.BlockSpec((3, 32, _BP), lambda i: (0, 0, i + _TCOFF)),
            pl.BlockSpec((32, 4, _BP), lambda i: (0, 0, i + _TCOFF)),
        ],
        out_specs=pl.BlockSpec((7, _BP), lambda i: (0, i)),
        out_shape=jax.ShapeDtypeStruct((7, _TCN), jnp.float32),
    )(dtT, tT, xyzT, rotT)


def kernel(xyz_motion, rotation_motion, t_idx, delta_t):
    N = xyz_motion.shape[0]
    xyzT = xyz_motion.transpose(2, 1, 0)
    rotT = rotation_motion.transpose(1, 2, 0)
    t1d = t_idx.reshape(N)
    dt1d = delta_t.reshape(N)
    sc_out = _sc_interp(xyzT, rotT, t1d, dt1d)
    tc_out = _tc_sweep(delta_t.reshape(1, N), t_idx.reshape(1, N), xyzT, rotT)
    out = jnp.concatenate([sc_out[:7], tc_out], axis=1)
    return out.transpose(1, 0)
